# baseline (speedup 1.0000x reference)
import jax
import jax.numpy as jnp
from jax.experimental import pallas as pl

NZ = 34
E = 8
MAX_QT = 15
MAX_SLI = 18
SCALE = 1.0
EP = 72
BT = 512


FX = 80


def _routed_kernel(qt_ref, sli_ref, ss_ref, eta_ref, gw_ref, gb_ref, out_ref):
    x = jnp.concatenate([
        qt_ref[...],
        jnp.zeros((40 - NZ, qt_ref.shape[1]), jnp.float32),
        sli_ref[...],
        ss_ref[...],
        jnp.zeros((FX - 76, qt_ref.shape[1]), jnp.float32),
    ], axis=0)
    y = jax.lax.dot_general(
        gw_ref[...], x, (((1,), (0,)), ((), ())),
        preferred_element_type=jnp.float32)
    y = y + gb_ref[...]
    eta = eta_ref[...]
    acc = jnp.zeros((EP, y.shape[1]), jnp.float32)
    for e in range(E):
        acc = acc + jnp.where(eta == e, y[e * EP:(e + 1) * EP, :], 0.0)
    out_ref[...] = acc[:2 * NZ, :]


def kernel(QT, SLI, SST, SOLIN, layer_mass, eta, W_base, b_base,
           coef_qt, int_qt, coef_sli, int_sli):
    nz, h, w = QT.shape
    n = h * w
    qt2 = QT.reshape(nz, n)
    sli2 = SLI.reshape(nz, n)
    ss2 = jnp.stack([SST.reshape(n), SOLIN.reshape(n)], axis=0)

    coef_cat = jnp.concatenate([coef_qt, coef_sli], axis=1)
    int_cat = jnp.concatenate([int_qt, int_sli], axis=1)
    npred = MAX_QT + MAX_SLI
    A = coef_cat[:, :, :npred]
    W_sel = jnp.concatenate([W_base[:MAX_QT], W_base[nz:nz + MAX_SLI]], axis=0)
    b_sel = jnp.concatenate([b_base[:MAX_QT], b_base[nz:nz + MAX_SLI]], axis=0)
    D = jnp.concatenate([
        coef_cat[:, :, npred + 1:npred + 1 + nz],
        coef_cat[:, :, npred + 1 + nz:npred + 1 + 2 * nz],
        coef_cat[:, :, npred:npred + 1],
        coef_cat[:, :, npred + 1 + 2 * nz:],
    ], axis=2)
    G = W_base[None] + SCALE * (jnp.einsum('eoc,cf->eof', A, W_sel) + D)
    g = b_base[None] + SCALE * (jnp.einsum('eoc,c->eo', A, b_sel) + int_cat)

    Gp = jnp.zeros((E, EP, FX), jnp.float32)
    Gp = Gp.at[:, :2 * nz, 0:nz].set(G[:, :, 0:nz])
    Gp = Gp.at[:, :2 * nz, 40:40 + nz].set(G[:, :, nz:2 * nz])
    Gp = Gp.at[:, :2 * nz, 74:76].set(G[:, :, 2 * nz:])
    Gp = Gp.reshape(E * EP, FX)
    gp = jnp.zeros((E, EP), jnp.float32).at[:, :2 * nz].set(g)
    gp = gp.reshape(E * EP, 1)
    eta2 = eta.reshape(1, n).astype(jnp.int32)

    out = pl.pallas_call(
        _routed_kernel,
        grid=(n // BT,),
        in_specs=[
            pl.BlockSpec((nz, BT), lambda i: (0, i)),
            pl.BlockSpec((nz, BT), lambda i: (0, i)),
            pl.BlockSpec((2, BT), lambda i: (0, i)),
            pl.BlockSpec((1, BT), lambda i: (0, i)),
            pl.BlockSpec((E * EP, FX), lambda i: (0, 0)),
            pl.BlockSpec((E * EP, 1), lambda i: (0, 0)),
        ],
        out_specs=pl.BlockSpec((2 * nz, BT), lambda i: (0, i)),
        out_shape=jax.ShapeDtypeStruct((2 * nz, n), jnp.float32),
    )(qt2, sli2, ss2, eta2, Gp, gp)
    return out.reshape(2, nz, h, w)


# speedup vs baseline: 1.2215x; 1.2215x over previous
import jax
import jax.numpy as jnp
from jax.experimental import pallas as pl
from jax.experimental.pallas import tpu as pltpu

NZ = 34
E = 8
MAX_QT = 15
MAX_SLI = 18
SCALE = 1.0
EP = 72
FX = 80
BT = 512


def _fold_weights(wb, bb, cq, cs, iq, isl):
    npred = MAX_QT + MAX_SLI
    coef_cat = jnp.concatenate([cq, cs], axis=1)
    int_cat = jnp.concatenate([iq, isl], axis=1)
    A = coef_cat[:, :, :npred]
    W_sel = jnp.concatenate([wb[:MAX_QT], wb[NZ:NZ + MAX_SLI]], axis=0)
    b_sel = jnp.concatenate([bb[:, :MAX_QT], bb[:, NZ:NZ + MAX_SLI]], axis=1)
    D = jnp.concatenate([
        coef_cat[:, :, npred + 1:npred + 1 + NZ],
        coef_cat[:, :, npred + 1 + NZ:npred + 1 + 2 * NZ],
        coef_cat[:, :, npred:npred + 1],
        coef_cat[:, :, npred + 1 + 2 * NZ:],
    ], axis=2)
    M = jax.lax.dot_general(
        A.reshape(E * 2 * NZ, npred), W_sel,
        (((1,), (0,)), ((), ())),
        preferred_element_type=jnp.float32).reshape(E, 2 * NZ, 70)
    R = wb[None] + SCALE * (M + D)
    g = bb + SCALE * (jnp.sum(A * b_sel[:, None, :], axis=-1) + int_cat)
    return R, g


def _routed_kernel(qt_ref, sli_ref, sst_ref, sol_ref, eta_ref,
                   wb_ref, bb_ref, cq_ref, iq_ref, cs_ref, is_ref,
                   out_ref, gs_ref):
    @pl.when(pl.program_id(0) == 0)
    def _fold():
        R, g = _fold_weights(wb_ref[...], bb_ref[...], cq_ref[...],
                             cs_ref[...], iq_ref[...], is_ref[...])
        gs_ref[...] = jnp.zeros((E * EP, FX), jnp.float32)
        for e in range(E):
            gs_ref[e * EP:e * EP + 2 * NZ, 0:NZ] = R[e, :, 0:NZ]
            gs_ref[e * EP:e * EP + 2 * NZ, 40:40 + NZ] = R[e, :, NZ:2 * NZ]
            gs_ref[e * EP:e * EP + 2 * NZ, 74:76] = R[e, :, 2 * NZ:]
            gs_ref[e * EP:e * EP + 2 * NZ, 76:77] = g[e, :, None]

    bt = qt_ref.shape[1]
    x = jnp.concatenate([
        qt_ref[...],
        jnp.zeros((40 - NZ, bt), jnp.float32),
        sli_ref[...],
        sst_ref[...],
        sol_ref[...],
        jnp.ones((1, bt), jnp.float32),
        jnp.zeros((FX - 77, bt), jnp.float32),
    ], axis=0)
    y = jax.lax.dot_general(
        gs_ref[...], x, (((1,), (0,)), ((), ())),
        preferred_element_type=jnp.float32)
    eta = eta_ref[...]
    acc = jnp.zeros((EP, bt), jnp.float32)
    for e in range(E):
        acc = acc + jnp.where(eta == e, y[e * EP:(e + 1) * EP, :], 0.0)
    out_ref[...] = acc[:2 * NZ, :]


def kernel(QT, SLI, SST, SOLIN, layer_mass, eta, W_base, b_base,
           coef_qt, int_qt, coef_sli, int_sli):
    nz, h, w = QT.shape
    n = h * w
    out = pl.pallas_call(
        _routed_kernel,
        grid=(n // BT,),
        in_specs=[
            pl.BlockSpec((nz, BT), lambda i: (0, i)),
            pl.BlockSpec((nz, BT), lambda i: (0, i)),
            pl.BlockSpec((1, BT), lambda i: (0, i)),
            pl.BlockSpec((1, BT), lambda i: (0, i)),
            pl.BlockSpec((1, BT), lambda i: (0, i)),
            pl.BlockSpec((2 * nz, 2 * nz + 2), lambda i: (0, 0)),
            pl.BlockSpec((1, 2 * nz), lambda i: (0, 0)),
            pl.BlockSpec((E, nz, 103), lambda i: (0, 0, 0)),
            pl.BlockSpec((E, nz), lambda i: (0, 0)),
            pl.BlockSpec((E, nz, 103), lambda i: (0, 0, 0)),
            pl.BlockSpec((E, nz), lambda i: (0, 0)),
        ],
        out_specs=pl.BlockSpec((2 * nz, BT), lambda i: (0, i)),
        out_shape=jax.ShapeDtypeStruct((2 * nz, n), jnp.float32),
        scratch_shapes=[pltpu.VMEM((E * EP, FX), jnp.float32)],
    )(QT.reshape(nz, n), SLI.reshape(nz, n), SST.reshape(1, n),
      SOLIN.reshape(1, n), eta.reshape(1, n),
      W_base, b_base.reshape(1, 2 * nz),
      coef_qt, int_qt, coef_sli, int_sli)
    return out.reshape(2, nz, h, w)


# speedup vs baseline: 1.7734x; 1.4519x over previous
import jax
import jax.numpy as jnp
from jax.experimental import pallas as pl
from jax.experimental.pallas import tpu as pltpu

NZ = 34
E = 8
MAX_QT = 15
MAX_SLI = 18
SCALE = 1.0
EP = 72
FX = 80
BT = 2048


def _fold_weights(wb, bb, cq, cs, iq, isl):
    npred = MAX_QT + MAX_SLI
    coef_cat = jnp.concatenate([cq, cs], axis=1)
    int_cat = jnp.concatenate([iq, isl], axis=1)
    A = coef_cat[:, :, :npred]
    W_sel = jnp.concatenate([wb[:MAX_QT], wb[NZ:NZ + MAX_SLI]], axis=0)
    b_sel = jnp.concatenate([bb[:, :MAX_QT], bb[:, NZ:NZ + MAX_SLI]], axis=1)
    D = jnp.concatenate([
        coef_cat[:, :, npred + 1:npred + 1 + NZ],
        coef_cat[:, :, npred + 1 + NZ:npred + 1 + 2 * NZ],
        coef_cat[:, :, npred:npred + 1],
        coef_cat[:, :, npred + 1 + 2 * NZ:],
    ], axis=2)
    M = jax.lax.dot_general(
        A.reshape(E * 2 * NZ, npred), W_sel,
        (((1,), (0,)), ((), ())),
        preferred_element_type=jnp.float32).reshape(E, 2 * NZ, 70)
    R = wb[None] + SCALE * (M + D)
    g = bb + SCALE * (jnp.sum(A * b_sel[:, None, :], axis=-1) + int_cat)
    return R, g


def _routed_kernel(qt_ref, sli_ref, sst_ref, sol_ref, eta_ref,
                   wb_ref, bb_ref, cq_ref, iq_ref, cs_ref, is_ref,
                   out_ref, gs_ref):
    @pl.when(pl.program_id(0) == 0)
    def _fold():
        R, g = _fold_weights(wb_ref[...], bb_ref[...], cq_ref[...],
                             cs_ref[...], iq_ref[...], is_ref[...])
        gs_ref[...] = jnp.zeros((E * EP, FX), jnp.float32)
        for e in range(E):
            gs_ref[e * EP:e * EP + 2 * NZ, 0:NZ] = R[e, :, 0:NZ]
            gs_ref[e * EP:e * EP + 2 * NZ, 40:40 + NZ] = R[e, :, NZ:2 * NZ]
            gs_ref[e * EP:e * EP + 2 * NZ, 74:76] = R[e, :, 2 * NZ:]
            gs_ref[e * EP:e * EP + 2 * NZ, 76:77] = g[e, :, None]

    bt = qt_ref.shape[1]
    x = jnp.concatenate([
        qt_ref[...],
        jnp.zeros((40 - NZ, bt), jnp.float32),
        sli_ref[...],
        sst_ref[...],
        sol_ref[...],
        jnp.ones((1, bt), jnp.float32),
        jnp.zeros((FX - 77, bt), jnp.float32),
    ], axis=0)
    y = jax.lax.dot_general(
        gs_ref[...], x, (((1,), (0,)), ((), ())),
        preferred_element_type=jnp.float32)
    eta = eta_ref[...]
    acc = jnp.zeros((EP, bt), jnp.float32)
    for e in range(E):
        acc = acc + jnp.where(eta == e, y[e * EP:(e + 1) * EP, :], 0.0)
    out_ref[...] = acc[:2 * NZ, :]


def kernel(QT, SLI, SST, SOLIN, layer_mass, eta, W_base, b_base,
           coef_qt, int_qt, coef_sli, int_sli):
    nz, h, w = QT.shape
    n = h * w
    out = pl.pallas_call(
        _routed_kernel,
        grid=(n // BT,),
        in_specs=[
            pl.BlockSpec((nz, BT), lambda i: (0, i)),
            pl.BlockSpec((nz, BT), lambda i: (0, i)),
            pl.BlockSpec((1, BT), lambda i: (0, i)),
            pl.BlockSpec((1, BT), lambda i: (0, i)),
            pl.BlockSpec((1, BT), lambda i: (0, i)),
            pl.BlockSpec((2 * nz, 2 * nz + 2), lambda i: (0, 0)),
            pl.BlockSpec((1, 2 * nz), lambda i: (0, 0)),
            pl.BlockSpec((E, nz, 103), lambda i: (0, 0, 0)),
            pl.BlockSpec((E, nz), lambda i: (0, 0)),
            pl.BlockSpec((E, nz, 103), lambda i: (0, 0, 0)),
            pl.BlockSpec((E, nz), lambda i: (0, 0)),
        ],
        out_specs=pl.BlockSpec((2 * nz, BT), lambda i: (0, i)),
        out_shape=jax.ShapeDtypeStruct((2 * nz, n), jnp.float32),
        scratch_shapes=[pltpu.VMEM((E * EP, FX), jnp.float32)],
    )(QT.reshape(nz, n), SLI.reshape(nz, n), SST.reshape(1, n),
      SOLIN.reshape(1, n), eta.reshape(1, n),
      W_base, b_base.reshape(1, 2 * nz),
      coef_qt, int_qt, coef_sli, int_sli)
    return out.reshape(2, nz, h, w)


# speedup vs baseline: 1.9112x; 1.0777x over previous
import jax
import jax.numpy as jnp
from jax.experimental import pallas as pl
from jax.experimental.pallas import tpu as pltpu

NZ = 34
E = 8
MAX_QT = 15
MAX_SLI = 18
SCALE = 1.0
EP = 72
FX = 80
BT = 4096


def _fold_weights(wb, bb, cq, cs, iq, isl):
    npred = MAX_QT + MAX_SLI
    coef_cat = jnp.concatenate([cq, cs], axis=1)
    int_cat = jnp.concatenate([iq, isl], axis=1)
    A = coef_cat[:, :, :npred]
    W_sel = jnp.concatenate([wb[:MAX_QT], wb[NZ:NZ + MAX_SLI]], axis=0)
    b_sel = jnp.concatenate([bb[:, :MAX_QT], bb[:, NZ:NZ + MAX_SLI]], axis=1)
    D = jnp.concatenate([
        coef_cat[:, :, npred + 1:npred + 1 + NZ],
        coef_cat[:, :, npred + 1 + NZ:npred + 1 + 2 * NZ],
        coef_cat[:, :, npred:npred + 1],
        coef_cat[:, :, npred + 1 + 2 * NZ:],
    ], axis=2)
    M = jax.lax.dot_general(
        A.reshape(E * 2 * NZ, npred), W_sel,
        (((1,), (0,)), ((), ())),
        preferred_element_type=jnp.float32).reshape(E, 2 * NZ, 70)
    R = wb[None] + SCALE * (M + D)
    g = bb + SCALE * (jnp.sum(A * b_sel[:, None, :], axis=-1) + int_cat)
    return R, g


def _routed_kernel(qt_ref, sli_ref, sst_ref, sol_ref, eta_ref,
                   wb_ref, bb_ref, cq_ref, iq_ref, cs_ref, is_ref,
                   out_ref, gs_ref):
    @pl.when(pl.program_id(0) == 0)
    def _fold():
        R, g = _fold_weights(wb_ref[...], bb_ref[...], cq_ref[...],
                             cs_ref[...], iq_ref[...], is_ref[...])
        gs_ref[...] = jnp.zeros((E * EP, FX), jnp.float32)
        for e in range(E):
            gs_ref[e * EP:e * EP + 2 * NZ, 0:NZ] = R[e, :, 0:NZ]
            gs_ref[e * EP:e * EP + 2 * NZ, 40:40 + NZ] = R[e, :, NZ:2 * NZ]
            gs_ref[e * EP:e * EP + 2 * NZ, 74:76] = R[e, :, 2 * NZ:]
            gs_ref[e * EP:e * EP + 2 * NZ, 76:77] = g[e, :, None]

    bt = qt_ref.shape[1]
    x = jnp.concatenate([
        qt_ref[...],
        jnp.zeros((40 - NZ, bt), jnp.float32),
        sli_ref[...],
        sst_ref[...],
        sol_ref[...],
        jnp.ones((1, bt), jnp.float32),
        jnp.zeros((FX - 77, bt), jnp.float32),
    ], axis=0)
    y = jax.lax.dot_general(
        gs_ref[...], x, (((1,), (0,)), ((), ())),
        preferred_element_type=jnp.float32)
    eta = eta_ref[...]
    acc = jnp.zeros((EP, bt), jnp.float32)
    for e in range(E):
        acc = acc + jnp.where(eta == e, y[e * EP:(e + 1) * EP, :], 0.0)
    out_ref[...] = acc[:2 * NZ, :]


def kernel(QT, SLI, SST, SOLIN, layer_mass, eta, W_base, b_base,
           coef_qt, int_qt, coef_sli, int_sli):
    nz, h, w = QT.shape
    n = h * w
    out = pl.pallas_call(
        _routed_kernel,
        grid=(n // BT,),
        in_specs=[
            pl.BlockSpec((nz, BT), lambda i: (0, i)),
            pl.BlockSpec((nz, BT), lambda i: (0, i)),
            pl.BlockSpec((1, BT), lambda i: (0, i)),
            pl.BlockSpec((1, BT), lambda i: (0, i)),
            pl.BlockSpec((1, BT), lambda i: (0, i)),
            pl.BlockSpec((2 * nz, 2 * nz + 2), lambda i: (0, 0)),
            pl.BlockSpec((1, 2 * nz), lambda i: (0, 0)),
            pl.BlockSpec((E, nz, 103), lambda i: (0, 0, 0)),
            pl.BlockSpec((E, nz), lambda i: (0, 0)),
            pl.BlockSpec((E, nz, 103), lambda i: (0, 0, 0)),
            pl.BlockSpec((E, nz), lambda i: (0, 0)),
        ],
        out_specs=pl.BlockSpec((2 * nz, BT), lambda i: (0, i)),
        out_shape=jax.ShapeDtypeStruct((2 * nz, n), jnp.float32),
        scratch_shapes=[pltpu.VMEM((E * EP, FX), jnp.float32)],
    )(QT.reshape(nz, n), SLI.reshape(nz, n), SST.reshape(1, n),
      SOLIN.reshape(1, n), eta.reshape(1, n),
      W_base, b_base.reshape(1, 2 * nz),
      coef_qt, int_qt, coef_sli, int_sli)
    return out.reshape(2, nz, h, w)


# speedup vs baseline: 1.9432x; 1.0167x over previous
import jax
import jax.numpy as jnp
from jax.experimental import pallas as pl
from jax.experimental.pallas import tpu as pltpu

NZ = 34
E = 8
MAX_QT = 15
MAX_SLI = 18
SCALE = 1.0
EP = 72
FX = 80
BT = 8192


def _fold_weights(wb, bb, cq, cs, iq, isl):
    npred = MAX_QT + MAX_SLI
    coef_cat = jnp.concatenate([cq, cs], axis=1)
    int_cat = jnp.concatenate([iq, isl], axis=1)
    A = coef_cat[:, :, :npred]
    W_sel = jnp.concatenate([wb[:MAX_QT], wb[NZ:NZ + MAX_SLI]], axis=0)
    b_sel = jnp.concatenate([bb[:, :MAX_QT], bb[:, NZ:NZ + MAX_SLI]], axis=1)
    D = jnp.concatenate([
        coef_cat[:, :, npred + 1:npred + 1 + NZ],
        coef_cat[:, :, npred + 1 + NZ:npred + 1 + 2 * NZ],
        coef_cat[:, :, npred:npred + 1],
        coef_cat[:, :, npred + 1 + 2 * NZ:],
    ], axis=2)
    M = jax.lax.dot_general(
        A.reshape(E * 2 * NZ, npred), W_sel,
        (((1,), (0,)), ((), ())),
        preferred_element_type=jnp.float32).reshape(E, 2 * NZ, 70)
    R = wb[None] + SCALE * (M + D)
    g = bb + SCALE * (jnp.sum(A * b_sel[:, None, :], axis=-1) + int_cat)
    return R, g


def _routed_kernel(qt_ref, sli_ref, sst_ref, sol_ref, eta_ref,
                   wb_ref, bb_ref, cq_ref, iq_ref, cs_ref, is_ref,
                   out_ref, gs_ref):
    @pl.when(pl.program_id(0) == 0)
    def _fold():
        R, g = _fold_weights(wb_ref[...], bb_ref[...], cq_ref[...],
                             cs_ref[...], iq_ref[...], is_ref[...])
        gs_ref[...] = jnp.zeros((E * EP, FX), jnp.float32)
        for e in range(E):
            gs_ref[e * EP:e * EP + 2 * NZ, 0:NZ] = R[e, :, 0:NZ]
            gs_ref[e * EP:e * EP + 2 * NZ, 40:40 + NZ] = R[e, :, NZ:2 * NZ]
            gs_ref[e * EP:e * EP + 2 * NZ, 74:76] = R[e, :, 2 * NZ:]
            gs_ref[e * EP:e * EP + 2 * NZ, 76:77] = g[e, :, None]

    bt = qt_ref.shape[1]
    x = jnp.concatenate([
        qt_ref[...],
        jnp.zeros((40 - NZ, bt), jnp.float32),
        sli_ref[...],
        sst_ref[...],
        sol_ref[...],
        jnp.ones((1, bt), jnp.float32),
        jnp.zeros((FX - 77, bt), jnp.float32),
    ], axis=0)
    y = jax.lax.dot_general(
        gs_ref[...], x, (((1,), (0,)), ((), ())),
        preferred_element_type=jnp.float32)
    eta = eta_ref[...]
    acc = jnp.zeros((EP, bt), jnp.float32)
    for e in range(E):
        acc = acc + jnp.where(eta == e, y[e * EP:(e + 1) * EP, :], 0.0)
    out_ref[...] = acc[:2 * NZ, :]


def kernel(QT, SLI, SST, SOLIN, layer_mass, eta, W_base, b_base,
           coef_qt, int_qt, coef_sli, int_sli):
    nz, h, w = QT.shape
    n = h * w
    out = pl.pallas_call(
        _routed_kernel,
        grid=(n // BT,),
        in_specs=[
            pl.BlockSpec((nz, BT), lambda i: (0, i)),
            pl.BlockSpec((nz, BT), lambda i: (0, i)),
            pl.BlockSpec((1, BT), lambda i: (0, i)),
            pl.BlockSpec((1, BT), lambda i: (0, i)),
            pl.BlockSpec((1, BT), lambda i: (0, i)),
            pl.BlockSpec((2 * nz, 2 * nz + 2), lambda i: (0, 0)),
            pl.BlockSpec((1, 2 * nz), lambda i: (0, 0)),
            pl.BlockSpec((E, nz, 103), lambda i: (0, 0, 0)),
            pl.BlockSpec((E, nz), lambda i: (0, 0)),
            pl.BlockSpec((E, nz, 103), lambda i: (0, 0, 0)),
            pl.BlockSpec((E, nz), lambda i: (0, 0)),
        ],
        out_specs=pl.BlockSpec((2 * nz, BT), lambda i: (0, i)),
        out_shape=jax.ShapeDtypeStruct((2 * nz, n), jnp.float32),
        scratch_shapes=[pltpu.VMEM((E * EP, FX), jnp.float32)],
    )(QT.reshape(nz, n), SLI.reshape(nz, n), SST.reshape(1, n),
      SOLIN.reshape(1, n), eta.reshape(1, n),
      W_base, b_base.reshape(1, 2 * nz),
      coef_qt, int_qt, coef_sli, int_sli)
    return out.reshape(2, nz, h, w)


# speedup vs baseline: 3.8524x; 1.9825x over previous
import jax
import jax.numpy as jnp
from jax.experimental import pallas as pl
from jax.experimental.pallas import tpu as pltpu

NZ = 34
E = 8
MAX_QT = 15
MAX_SLI = 18
SCALE = 1.0
EP = 72
FX = 80
BT = 8192


def _fold_weights(wb, bb, cq, cs, iq, isl):
    npred = MAX_QT + MAX_SLI
    coef_cat = jnp.concatenate([cq, cs], axis=1)
    int_cat = jnp.concatenate([iq, isl], axis=1)
    A = coef_cat[:, :, :npred]
    W_sel = jnp.concatenate([wb[:MAX_QT], wb[NZ:NZ + MAX_SLI]], axis=0)
    b_sel = jnp.concatenate([bb[:, :MAX_QT], bb[:, NZ:NZ + MAX_SLI]], axis=1)
    D = jnp.concatenate([
        coef_cat[:, :, npred + 1:npred + 1 + NZ],
        coef_cat[:, :, npred + 1 + NZ:npred + 1 + 2 * NZ],
        coef_cat[:, :, npred:npred + 1],
        coef_cat[:, :, npred + 1 + 2 * NZ:],
    ], axis=2)
    M = jax.lax.dot_general(
        A.reshape(E * 2 * NZ, npred), W_sel,
        (((1,), (0,)), ((), ())),
        preferred_element_type=jnp.float32).reshape(E, 2 * NZ, 70)
    R = wb[None] + SCALE * (M + D)
    g = bb + SCALE * (jnp.sum(A * b_sel[:, None, :], axis=-1) + int_cat)
    return R, g


def _routed_kernel(qt_ref, sli_ref, sst_ref, sol_ref, eta_ref,
                   wb_ref, bb_ref, cq_ref, iq_ref, cs_ref, is_ref,
                   out_ref, gs_ref):
    @pl.when(pl.program_id(0) == 0)
    def _fold():
        R, g = _fold_weights(wb_ref[...], bb_ref[...], cq_ref[...],
                             cs_ref[...], iq_ref[...], is_ref[...])
        gs_ref[...] = jnp.zeros((E * EP, FX), jnp.float32)
        for e in range(E):
            gs_ref[e * EP:e * EP + 2 * NZ, 0:NZ] = R[e, :, 0:NZ]
            gs_ref[e * EP:e * EP + 2 * NZ, 40:40 + NZ] = R[e, :, NZ:2 * NZ]
            gs_ref[e * EP:e * EP + 2 * NZ, 74:76] = R[e, :, 2 * NZ:]
            gs_ref[e * EP:e * EP + 2 * NZ, 76:77] = g[e, :, None]

    bt = qt_ref.shape[1] * qt_ref.shape[2]
    x = jnp.concatenate([
        qt_ref[...].reshape(NZ, bt),
        jnp.zeros((40 - NZ, bt), jnp.float32),
        sli_ref[...].reshape(NZ, bt),
        sst_ref[...],
        sol_ref[...],
        jnp.ones((1, bt), jnp.float32),
        jnp.zeros((FX - 77, bt), jnp.float32),
    ], axis=0)
    y = jax.lax.dot_general(
        gs_ref[...], x, (((1,), (0,)), ((), ())),
        preferred_element_type=jnp.float32)
    eta = eta_ref[...]
    acc = jnp.zeros((EP, bt), jnp.float32)
    for e in range(E):
        acc = acc + jnp.where(eta == e, y[e * EP:(e + 1) * EP, :], 0.0)
    out_ref[...] = acc[:2 * NZ, :].reshape(out_ref.shape)


def kernel(QT, SLI, SST, SOLIN, layer_mass, eta, W_base, b_base,
           coef_qt, int_qt, coef_sli, int_sli):
    nz, h, w = QT.shape
    n = h * w
    hb = BT // w
    out = pl.pallas_call(
        _routed_kernel,
        grid=(n // BT,),
        in_specs=[
            pl.BlockSpec((nz, hb, w), lambda i: (0, i, 0)),
            pl.BlockSpec((nz, hb, w), lambda i: (0, i, 0)),
            pl.BlockSpec((1, BT), lambda i: (0, i)),
            pl.BlockSpec((1, BT), lambda i: (0, i)),
            pl.BlockSpec((1, BT), lambda i: (0, i)),
            pl.BlockSpec((2 * nz, 2 * nz + 2), lambda i: (0, 0)),
            pl.BlockSpec((1, 2 * nz), lambda i: (0, 0)),
            pl.BlockSpec((E, nz, 103), lambda i: (0, 0, 0)),
            pl.BlockSpec((E, nz), lambda i: (0, 0)),
            pl.BlockSpec((E, nz, 103), lambda i: (0, 0, 0)),
            pl.BlockSpec((E, nz), lambda i: (0, 0)),
        ],
        out_specs=pl.BlockSpec((2, nz, hb, w), lambda i: (0, 0, i, 0)),
        out_shape=jax.ShapeDtypeStruct((2, nz, h, w), jnp.float32),
        scratch_shapes=[pltpu.VMEM((E * EP, FX), jnp.float32)],
    )(QT, SLI, SST.reshape(1, n),
      SOLIN.reshape(1, n), eta.reshape(1, n),
      W_base, b_base.reshape(1, 2 * nz),
      coef_qt, int_qt, coef_sli, int_sli)
    return out


# speedup vs baseline: 4.7894x; 1.2433x over previous
import jax
import jax.numpy as jnp
from jax.experimental import pallas as pl
from jax.experimental.pallas import tpu as pltpu

NZ = 34
E = 8
MAX_QT = 15
MAX_SLI = 18
SCALE = 1.0
EP = 72
FX = 80
BT = 8192


def _fold_weights(wb, bb, cq, cs, iq, isl):
    npred = MAX_QT + MAX_SLI
    coef_cat = jnp.concatenate([cq, cs], axis=1)
    int_cat = jnp.concatenate([iq, isl], axis=1)
    A = coef_cat[:, :, :npred]
    W_sel = jnp.concatenate([wb[:MAX_QT], wb[NZ:NZ + MAX_SLI]], axis=0)
    b_sel = jnp.concatenate([bb[:, :MAX_QT], bb[:, NZ:NZ + MAX_SLI]], axis=1)
    D = jnp.concatenate([
        coef_cat[:, :, npred + 1:npred + 1 + NZ],
        coef_cat[:, :, npred + 1 + NZ:npred + 1 + 2 * NZ],
        coef_cat[:, :, npred:npred + 1],
        coef_cat[:, :, npred + 1 + 2 * NZ:],
    ], axis=2)
    M = jax.lax.dot_general(
        A.reshape(E * 2 * NZ, npred), W_sel,
        (((1,), (0,)), ((), ())),
        preferred_element_type=jnp.float32).reshape(E, 2 * NZ, 70)
    R = wb[None] + SCALE * (M + D)
    g = bb + SCALE * (jnp.sum(A * b_sel[:, None, :], axis=-1) + int_cat)
    return R, g


def _routed_kernel(qt_ref, sli_ref, sst_ref, sol_ref, eta_ref,
                   wb_ref, bb_ref, cq_ref, iq_ref, cs_ref, is_ref,
                   out_ref, gs_ref):
    @pl.when(pl.program_id(0) == 0)
    def _fold():
        R, g = _fold_weights(wb_ref[...], bb_ref[...], cq_ref[...],
                             cs_ref[...], iq_ref[...], is_ref[...])
        gs_ref[...] = jnp.zeros((E * EP, FX), jnp.float32)
        for e in range(E):
            gs_ref[e * EP:e * EP + 2 * NZ, 0:NZ] = R[e, :, 0:NZ]
            gs_ref[e * EP:e * EP + 2 * NZ, 40:40 + NZ] = R[e, :, NZ:2 * NZ]
            gs_ref[e * EP:e * EP + 2 * NZ, 74:76] = R[e, :, 2 * NZ:]
            gs_ref[e * EP:e * EP + 2 * NZ, 76:77] = g[e, :, None]

    bt = qt_ref.shape[1] * qt_ref.shape[2]
    x = jnp.concatenate([
        qt_ref[...].reshape(NZ, bt),
        jnp.zeros((40 - NZ, bt), jnp.float32),
        sli_ref[...].reshape(NZ, bt),
        sst_ref[...].reshape(1, bt),
        sol_ref[...].reshape(1, bt),
        jnp.ones((1, bt), jnp.float32),
        jnp.zeros((FX - 77, bt), jnp.float32),
    ], axis=0)
    y = jax.lax.dot_general(
        gs_ref[...], x, (((1,), (0,)), ((), ())),
        preferred_element_type=jnp.float32)
    eta = eta_ref[...].reshape(1, bt)
    acc = jnp.zeros((EP, bt), jnp.float32)
    for e in range(E):
        acc = acc + jnp.where(eta == e, y[e * EP:(e + 1) * EP, :], 0.0)
    out_ref[...] = acc[:2 * NZ, :].reshape(out_ref.shape)


def kernel(QT, SLI, SST, SOLIN, layer_mass, eta, W_base, b_base,
           coef_qt, int_qt, coef_sli, int_sli):
    nz, h, w = QT.shape
    n = h * w
    hb = BT // w
    out = pl.pallas_call(
        _routed_kernel,
        grid=(n // BT,),
        in_specs=[
            pl.BlockSpec((nz, hb, w), lambda i: (0, i, 0)),
            pl.BlockSpec((nz, hb, w), lambda i: (0, i, 0)),
            pl.BlockSpec((hb, w), lambda i: (i, 0)),
            pl.BlockSpec((hb, w), lambda i: (i, 0)),
            pl.BlockSpec((hb, w), lambda i: (i, 0)),
            pl.BlockSpec((2 * nz, 2 * nz + 2), lambda i: (0, 0)),
            pl.BlockSpec((1, 2 * nz), lambda i: (0, 0)),
            pl.BlockSpec((E, nz, 103), lambda i: (0, 0, 0)),
            pl.BlockSpec((E, nz), lambda i: (0, 0)),
            pl.BlockSpec((E, nz, 103), lambda i: (0, 0, 0)),
            pl.BlockSpec((E, nz), lambda i: (0, 0)),
        ],
        out_specs=pl.BlockSpec((2, nz, hb, w), lambda i: (0, 0, i, 0)),
        out_shape=jax.ShapeDtypeStruct((2, nz, h, w), jnp.float32),
        scratch_shapes=[pltpu.VMEM((E * EP, FX), jnp.float32)],
    )(QT, SLI, SST, SOLIN, eta,
      W_base, b_base.reshape(1, 2 * nz),
      coef_qt, int_qt, coef_sli, int_sli)
    return out
